# baseline (speedup 1.0000x reference)
import functools

import jax
import jax.numpy as jnp
from jax import lax
from jax.experimental import pallas as pl
from jax.experimental.pallas import tpu as pltpu
from jax.experimental.pallas import tpu_sc as plsc

_LENGTH = 1000000
_FEAT = 64
_NCLS = 1000
_B = 16384
_MOM = 0.5

_NC = 2
_NS = 16
_NW = _NC * _NS
_CH = _B // _NW
_PIECE = 128
_NP = _CH // _PIECE
_RD = 4
_LANES = 16


def _worker_base():
  wid = lax.axis_index("s") * _NC + lax.axis_index("c")
  return wid * _CH


def _shift_piece(dst_row, src_row, off):
  for k in range(_PIECE // _LANES):
    sl = pl.ds(k * _LANES, _LANES)
    dst_row[sl] = src_row[sl] + off


def _feature_stream_loop(j, idx2, ring, dummy_hbm, sem, issue, row_scale):
  def wait_one():
    pltpu.make_async_copy(dummy_hbm.at[pl.ds(0, _PIECE)], ring.at[0],
                          sem).wait()

  def body(f, _):
    @pl.when(f >= _RD)
    def _():
      wait_one()
    m = lax.rem(f, _RD)
    _shift_piece(ring.at[m], idx2.at[j], f * row_scale)
    issue(f, ring.at[m])
    return 0

  lax.fori_loop(0, _FEAT, body, 0, unroll=False)
  def drain(_, __):
    wait_one()
    return 0
  lax.fori_loop(0, _RD, drain, 0, unroll=False)


def _sc_gather_body(ind_hbm, fb1d, lb_hbm, fold_hbm, lold_hbm,
                    idx2, ring, stage, lab2, sem, gsem):
  base = _worker_base()
  for j in range(_NP):
    pltpu.sync_copy(ind_hbm.at[pl.ds(base + j * _PIECE, _PIECE)], idx2.at[j])
  for j in range(_NP):
    pltpu.async_copy(lb_hbm.at[idx2.at[j]], lab2.at[j], sem).wait()
    pltpu.sync_copy(lab2.at[j], lold_hbm.at[pl.ds(base + j * _PIECE, _PIECE)])

  for j in range(_NP):
    def issue(f, idx_ref, j=j):
      pltpu.async_copy(fb1d.at[idx_ref],
                       stage.at[f, pl.ds(j * _PIECE, _PIECE)], gsem)
    _feature_stream_loop(j, idx2, ring, ind_hbm, gsem, issue, _LENGTH)

  pltpu.sync_copy(stage, fold_hbm.at[:, pl.ds(base, _CH)])


@functools.cache
def _get_sc_gather():
  return pl.kernel(
      _sc_gather_body,
      out_type=(jax.ShapeDtypeStruct((_FEAT, _B), jnp.float32),
                jax.ShapeDtypeStruct((_B,), jnp.int32)),
      mesh=plsc.VectorSubcoreMesh(core_axis_name="c", subcore_axis_name="s"),
      compiler_params=pltpu.CompilerParams(use_tc_tiling_on_sc=False),
      scratch_types=[
          pltpu.VMEM((_NP, _PIECE), jnp.int32),
          pltpu.VMEM((_RD, _PIECE), jnp.int32),
          pltpu.VMEM((_FEAT, _CH), jnp.float32),
          pltpu.VMEM((_NP, _PIECE), jnp.int32),
          pltpu.SemaphoreType.DMA,
          pltpu.SemaphoreType.DMA,
      ],
  )


def _sc_scatter_body(ind_hbm, win_hbm, v2flat, nl_hbm, fb1d, lb_hbm,
                     idx2, src2, ring, vstage, lab2, sem, gsem):
  base = _worker_base()
  for j in range(_NP):
    pltpu.sync_copy(ind_hbm.at[pl.ds(base + j * _PIECE, _PIECE)], idx2.at[j])
  for j in range(_NP):
    pltpu.async_copy(win_hbm.at[idx2.at[j]], src2.at[j], sem).wait()
  for j in range(_NP):
    pltpu.async_copy(nl_hbm.at[src2.at[j]], lab2.at[j], sem).wait()
  for j in range(_NP):
    pltpu.async_copy(lab2.at[j], lb_hbm.at[idx2.at[j]], sem).wait()

  for j in range(_NP):
    def issue_v(f, idx_ref, j=j):
      pltpu.async_copy(v2flat.at[idx_ref],
                       vstage.at[f, pl.ds(j * _PIECE, _PIECE)], gsem)
    _feature_stream_loop(j, src2, ring, ind_hbm, gsem, issue_v, _B)
  for j in range(_NP):
    def issue_s(f, idx_ref, j=j):
      pltpu.async_copy(vstage.at[f, pl.ds(j * _PIECE, _PIECE)],
                       fb1d.at[idx_ref], gsem)
    _feature_stream_loop(j, idx2, ring, ind_hbm, gsem, issue_s, _LENGTH)


@functools.cache
def _get_sc_scatter():
  return pl.kernel(
      _sc_scatter_body,
      out_type=(),
      mesh=plsc.VectorSubcoreMesh(core_axis_name="c", subcore_axis_name="s"),
      compiler_params=pltpu.CompilerParams(use_tc_tiling_on_sc=False),
      scratch_types=[
          pltpu.VMEM((_NP, _PIECE), jnp.int32),
          pltpu.VMEM((_NP, _PIECE), jnp.int32),
          pltpu.VMEM((_RD, _PIECE), jnp.int32),
          pltpu.VMEM((_FEAT, _CH), jnp.float32),
          pltpu.VMEM((_NP, _PIECE), jnp.int32),
          pltpu.SemaphoreType.DMA,
          pltpu.SemaphoreType.DMA,
      ],
  )


_TC_BLK = 1024
_TC_GRID = _B // _TC_BLK


def _tc_body(feat_ref, fold_ref, lold_ref, cents_ref, v2_ref, nl_ref, cnt_ref):
  i = pl.program_id(0)
  f = feat_ref[...]
  fo = fold_ref[...]
  fn = f / (jnp.sqrt(jnp.sum(f * f, axis=0, keepdims=True)) + 1e-10)
  fnew = (1.0 - _MOM) * fo + _MOM * fn
  v2 = fnew / (jnp.sqrt(jnp.sum(fnew * fnew, axis=0, keepdims=True)) + 1e-10)
  v2_ref[...] = v2
  sim = lax.dot_general(cents_ref[...], v2, (((1,), (0,)), ((), ())),
                        preferred_element_type=jnp.float32)
  m = jnp.max(sim, axis=0, keepdims=True)
  cls_iota = lax.broadcasted_iota(jnp.int32, sim.shape, 0)
  lbl = jnp.min(jnp.where(sim >= m, cls_iota, _NCLS), axis=0).astype(jnp.int32)
  nl_ref[...] = lbl
  changed = jnp.sum((lbl != lold_ref[...]).astype(jnp.float32))

  @pl.when(i == 0)
  def _():
    cnt_ref[0, 0] = 0.0

  cnt_ref[0, 0] += changed


_tc_compute = pl.pallas_call(
    _tc_body,
    grid=(_TC_GRID,),
    in_specs=[
        pl.BlockSpec((_FEAT, _TC_BLK), lambda i: (0, i)),
        pl.BlockSpec((_FEAT, _TC_BLK), lambda i: (0, i)),
        pl.BlockSpec((_TC_BLK,), lambda i: (i,)),
        pl.BlockSpec((_NCLS, _FEAT), lambda i: (0, 0)),
    ],
    out_specs=[
        pl.BlockSpec((_FEAT, _TC_BLK), lambda i: (0, i)),
        pl.BlockSpec((_TC_BLK,), lambda i: (i,)),
        pl.BlockSpec(memory_space=pltpu.SMEM, block_shape=(1, 1),
                     index_map=lambda i: (0, 0)),
    ],
    out_shape=[
        jax.ShapeDtypeStruct((_FEAT, _B), jnp.float32),
        jax.ShapeDtypeStruct((_B,), jnp.int32),
        jax.ShapeDtypeStruct((1, 1), jnp.float32),
    ],
)


def kernel(ind, feature, feature_bank, label_bank, centroids):
  ind = ind.astype(jnp.int32)
  fb_ref = jax.new_ref(feature_bank.swapaxes(0, 1).reshape(_FEAT * _LENGTH))
  lb_ref = jax.new_ref(label_bank)
  fold_t, lold = _get_sc_gather()(ind, fb_ref, lb_ref)
  feat_t = feature.swapaxes(0, 1)
  v2t, nl, cnt = _tc_compute(feat_t, fold_t, lold, centroids)
  pos = jnp.arange(_B, dtype=jnp.int32)
  win = jnp.zeros((_LENGTH,), jnp.int32).at[ind].max(pos)
  _get_sc_scatter()(ind, win, v2t.reshape(_FEAT * _B), nl, fb_ref, lb_ref)
  new_fb = fb_ref[...].reshape(_FEAT, _LENGTH).swapaxes(0, 1)
  change_ratio = cnt[0, 0] * (1.0 / _B)
  return change_ratio, new_fb, lb_ref[...]


# speedup vs baseline: 9.4189x; 9.4189x over previous
import functools

import jax
import jax.numpy as jnp
from jax import lax
from jax.experimental import pallas as pl
from jax.experimental.pallas import tpu as pltpu
from jax.experimental.pallas import tpu_sc as plsc

_LENGTH = 1000000
_FEAT = 64
_NCLS = 1000
_B = 16384
_MOM = 0.5

_NC = 2
_NS = 16
_NW = _NC * _NS
_CH = _B // _NW
_PIECE = 128
_NP = _CH // _PIECE

def _worker_base():
  wid = lax.axis_index("s") * _NC + lax.axis_index("c")
  return wid * _CH


def _sc_gather_body(ind_hbm, fbank_hbm, lbank_hbm, fold_hbm, lold_hbm,
                    idx2, rows_v, lab2, sem):
  base = _worker_base()
  for j in range(_NP):
    pltpu.sync_copy(ind_hbm.at[pl.ds(base + j * _PIECE, _PIECE)], idx2.at[j])
  for j in range(_NP):
    pltpu.async_copy(fbank_hbm.at[idx2.at[j]],
                     rows_v.at[pl.ds(j * _PIECE, _PIECE), :], sem).wait()
    pltpu.async_copy(lbank_hbm.at[idx2.at[j]], lab2.at[j], sem).wait()
  pltpu.sync_copy(rows_v, fold_hbm.at[pl.ds(base, _CH)])
  for j in range(_NP):
    pltpu.sync_copy(lab2.at[j], lold_hbm.at[pl.ds(base + j * _PIECE, _PIECE)])


@functools.cache
def _get_sc_gather():
  return pl.kernel(
      _sc_gather_body,
      out_type=(jax.ShapeDtypeStruct((_B, _FEAT), jnp.float32),
                jax.ShapeDtypeStruct((_B,), jnp.int32)),
      mesh=plsc.VectorSubcoreMesh(core_axis_name="c", subcore_axis_name="s"),
      compiler_params=pltpu.CompilerParams(use_tc_tiling_on_sc=False),
      scratch_types=[
          pltpu.VMEM((_NP, _PIECE), jnp.int32),
          pltpu.VMEM((_CH, _FEAT), jnp.float32),
          pltpu.VMEM((_NP, _PIECE), jnp.int32),
          pltpu.SemaphoreType.DMA,
      ],
  )


def _sc_scatter_body(ind_hbm, win_hbm, v2_hbm, nl_hbm, fb_ref, lb_ref,
                     idx2, src2, rows_v, lab2, sem):
  base = _worker_base()
  for j in range(_NP):
    pltpu.sync_copy(ind_hbm.at[pl.ds(base + j * _PIECE, _PIECE)], idx2.at[j])
  for j in range(_NP):
    pltpu.async_copy(win_hbm.at[idx2.at[j]], src2.at[j], sem).wait()
  for j in range(_NP):
    pltpu.async_copy(v2_hbm.at[src2.at[j]],
                     rows_v.at[pl.ds(j * _PIECE, _PIECE), :], sem).wait()
    pltpu.async_copy(nl_hbm.at[src2.at[j]], lab2.at[j], sem).wait()
  for j in range(_NP):
    pltpu.async_copy(rows_v.at[pl.ds(j * _PIECE, _PIECE), :],
                     fb_ref.at[idx2.at[j]], sem).wait()
    pltpu.async_copy(lab2.at[j], lb_ref.at[idx2.at[j]], sem).wait()


@functools.cache
def _get_sc_scatter():
  return pl.kernel(
      _sc_scatter_body,
      out_type=(),
      mesh=plsc.VectorSubcoreMesh(core_axis_name="c", subcore_axis_name="s"),
      compiler_params=pltpu.CompilerParams(use_tc_tiling_on_sc=False),
      scratch_types=[
          pltpu.VMEM((_NP, _PIECE), jnp.int32),
          pltpu.VMEM((_NP, _PIECE), jnp.int32),
          pltpu.VMEM((_CH, _FEAT), jnp.float32),
          pltpu.VMEM((_NP, _PIECE), jnp.int32),
          pltpu.SemaphoreType.DMA,
      ],
  )


_TC_BLK = 1024
_TC_GRID = _B // _TC_BLK


def _tc_body(feat_ref, fold_ref, lold_ref, cents_ref, v2_ref, nl_ref, cnt_ref):
  i = pl.program_id(0)
  f = feat_ref[...]
  fo = fold_ref[...]
  fn = f / (jnp.sqrt(jnp.sum(f * f, axis=1, keepdims=True)) + 1e-10)
  fnew = (1.0 - _MOM) * fo + _MOM * fn
  v2 = fnew / (jnp.sqrt(jnp.sum(fnew * fnew, axis=1, keepdims=True)) + 1e-10)
  v2_ref[...] = v2
  sim = lax.dot_general(v2, cents_ref[...], (((1,), (1,)), ((), ())),
                        preferred_element_type=jnp.float32)
  m = jnp.max(sim, axis=1, keepdims=True)
  cls_iota = lax.broadcasted_iota(jnp.int32, sim.shape, 1)
  lbl = jnp.min(jnp.where(sim >= m, cls_iota, _NCLS), axis=1).astype(jnp.int32)
  nl_ref[...] = lbl
  changed = jnp.sum((lbl != lold_ref[...]).astype(jnp.float32))

  @pl.when(i == 0)
  def _():
    cnt_ref[0, 0] = 0.0

  cnt_ref[0, 0] += changed


_tc_compute = pl.pallas_call(
    _tc_body,
    grid=(_TC_GRID,),
    in_specs=[
        pl.BlockSpec((_TC_BLK, _FEAT), lambda i: (i, 0)),
        pl.BlockSpec((_TC_BLK, _FEAT), lambda i: (i, 0)),
        pl.BlockSpec((_TC_BLK,), lambda i: (i,)),
        pl.BlockSpec((_NCLS, _FEAT), lambda i: (0, 0)),
    ],
    out_specs=[
        pl.BlockSpec((_TC_BLK, _FEAT), lambda i: (i, 0)),
        pl.BlockSpec((_TC_BLK,), lambda i: (i,)),
        pl.BlockSpec(memory_space=pltpu.SMEM, block_shape=(1, 1),
                     index_map=lambda i: (0, 0)),
    ],
    out_shape=[
        jax.ShapeDtypeStruct((_B, _FEAT), jnp.float32),
        jax.ShapeDtypeStruct((_B,), jnp.int32),
        jax.ShapeDtypeStruct((1, 1), jnp.float32),
    ],
)


def kernel(ind, feature, feature_bank, label_bank, centroids):
  ind = ind.astype(jnp.int32)
  fb_ref = jax.new_ref(feature_bank)
  lb_ref = jax.new_ref(label_bank)
  fold, lold = _get_sc_gather()(ind, fb_ref, lb_ref)
  v2, nl, cnt = _tc_compute(feature, fold, lold, centroids)
  pos = jnp.arange(_B, dtype=jnp.int32)
  win = jnp.zeros((_LENGTH,), jnp.int32).at[ind].max(pos)
  _get_sc_scatter()(ind, win, v2, nl, fb_ref, lb_ref)
  change_ratio = cnt[0, 0] * (1.0 / _B)
  return change_ratio, fb_ref[...], lb_ref[...]
